# baseline (speedup 1.0000x reference)
import functools

import jax
import jax.numpy as jnp
from jax import lax
from jax.experimental import pallas as pl
from jax.experimental.pallas import tpu as pltpu
from jax.experimental.pallas import tpu_sc as plsc

NC = 2
NS = 16
L = 16
NW = NC * NS
B = 16384
F = 64
BPW = B // NW
CH = 128
NCH = BPW // CH
G = BPW // L

_mesh = plsc.VectorSubcoreMesh(core_axis_name="c", subcore_axis_name="s")


@functools.partial(
    pl.kernel,
    out_type=jax.ShapeDtypeStruct((B,), jnp.float32),
    mesh=_mesh,
    compiler_params=pltpu.CompilerParams(needs_layout_passes=False,
                                         use_tc_tiling_on_sc=False),
    scratch_types=[
        pltpu.VMEM((NCH, CH), jnp.int32),
        pltpu.VMEM((NCH, CH), jnp.int32),
        pltpu.VMEM((BPW, F), jnp.float32),
        pltpu.VMEM((BPW, F), jnp.float32),
        pltpu.VMEM((BPW,), jnp.float32),
        pltpu.VMEM((BPW,), jnp.float32),
        pltpu.VMEM((BPW,), jnp.float32),
        pltpu.SemaphoreType.DMA,
    ],
)
def _sc_predict(x1, x2, U, M, UB, MB, out,
                idx1, idx2, urows, mrows, ubv, mbv, outv, sem):
    wid = lax.axis_index("s") * NC + lax.axis_index("c")
    base = wid * BPW

    pltpu.sync_copy(x1.at[pl.ds(wid * NCH, NCH)], idx1)
    pltpu.sync_copy(x2.at[pl.ds(wid * NCH, NCH)], idx2)

    copies = []
    for j in range(NCH):
        sl = pl.ds(j * CH, CH)
        copies.append(pltpu.async_copy(U.at[idx1.at[j]], urows.at[sl], sem))
        copies.append(pltpu.async_copy(M.at[idx2.at[j]], mrows.at[sl], sem))
        copies.append(pltpu.async_copy(UB.at[idx1.at[j]], ubv.at[sl], sem))
        copies.append(pltpu.async_copy(MB.at[idx2.at[j]], mbv.at[sl], sem))
    for c in copies:
        c.wait()

    lanes = lax.iota(jnp.int32, L)

    def group(g, carry):
        p = jnp.zeros((L,), jnp.float32)
        for r in range(L):
            row = g * L + r
            acc = urows[row, pl.ds(0, L)] * mrows[row, pl.ds(0, L)]
            for j in range(1, F // L):
                acc = acc + (urows[row, pl.ds(j * L, L)]
                             * mrows[row, pl.ds(j * L, L)])
            p = jnp.where(lanes == r, jnp.sum(acc), p)
        p = p + ubv[pl.ds(g * L, L)] + mbv[pl.ds(g * L, L)]
        p = jnp.minimum(jnp.maximum(p, 0.0), 5.0)
        outv[pl.ds(g * L, L)] = p
        return carry

    lax.fori_loop(0, G, group, 0)
    pltpu.sync_copy(outv, out.at[pl.ds(base, BPW)])


def kernel(x1, x2, U, M, UB, MB):
    x1r = x1.astype(jnp.int32).reshape(NW * NCH, CH)
    x2r = x2.astype(jnp.int32).reshape(NW * NCH, CH)
    return _sc_predict(x1r, x2r, U, M, UB.reshape(-1), MB.reshape(-1))


# speedup vs baseline: 1.4666x; 1.4666x over previous
import functools

import jax
import jax.numpy as jnp
from jax import lax
from jax.experimental import pallas as pl
from jax.experimental.pallas import tpu as pltpu
from jax.experimental.pallas import tpu_sc as plsc

NC = 2
NS = 16
L = 16
NW = NC * NS
B = 16384
F = 64
BPW = B // NW
CH = 128
NCH = BPW // CH
RC = 128
NRC = BPW // RC

_mesh = plsc.VectorSubcoreMesh(core_axis_name="c", subcore_axis_name="s")


@functools.partial(
    pl.kernel,
    out_type=jax.ShapeDtypeStruct((B,), jnp.float32),
    mesh=_mesh,
    compiler_params=pltpu.CompilerParams(needs_layout_passes=False),
    scratch_types=[
        pltpu.VMEM((BPW,), jnp.int32),
        pltpu.VMEM((BPW,), jnp.int32),
        pltpu.VMEM((RC, F), jnp.float32),
        pltpu.VMEM((RC, F), jnp.float32),
        pltpu.VMEM((BPW,), jnp.float32),
        pltpu.VMEM((BPW,), jnp.float32),
        pltpu.VMEM((BPW,), jnp.float32),
        pltpu.SemaphoreType.DMA,
        pltpu.SemaphoreType.DMA,
    ],
)
def _sc_predict(x1, x2, U, M, UB, MB, out,
                idx1, idx2, ur, mr, ubv, mbv, outv,
                sem, rsem):
    wid = lax.axis_index("s") * NC + lax.axis_index("c")
    base = wid * BPW

    pltpu.sync_copy(x1.at[pl.ds(base, BPW)], idx1)
    pltpu.sync_copy(x2.at[pl.ds(base, BPW)], idx2)

    copies = []
    for j in range(NCH):
        sl = pl.ds(j * CH, CH)
        copies.append(pltpu.async_copy(UB.at[idx1.at[sl]], ubv.at[sl], sem))
        copies.append(pltpu.async_copy(MB.at[idx2.at[sl]], mbv.at[sl], sem))

    lanes = lax.iota(jnp.int32, L)

    for c in range(NRC):
        cbase = c * RC

        def fetch_group(g, carry):
            uvec = idx1[pl.ds(cbase + g * L, L)]
            mvec = idx2[pl.ds(cbase + g * L, L)]
            for r in range(L):
                i = g * L + r
                pltpu.async_copy(U.at[pl.ds(uvec[r], 1)],
                                 ur.at[pl.ds(i, 1)], rsem)
                pltpu.async_copy(M.at[pl.ds(mvec[r], 1)],
                                 mr.at[pl.ds(i, 1)], rsem)
            return carry

        lax.fori_loop(0, RC // L, fetch_group, 0)

        def drain_row(i, carry):
            pltpu.make_async_copy(U.at[pl.ds(0, 1)], ur.at[pl.ds(0, 1)],
                                  rsem).wait()
            pltpu.make_async_copy(M.at[pl.ds(0, 1)], mr.at[pl.ds(0, 1)],
                                  rsem).wait()
            return carry

        lax.fori_loop(0, RC, drain_row, 0)

        def group(g, carry):
            p = jnp.zeros((L,), jnp.float32)
            for r in range(L):
                row = g * L + r
                acc = ur[row, pl.ds(0, L)] * mr[row, pl.ds(0, L)]
                for j in range(1, F // L):
                    acc = acc + (ur[row, pl.ds(j * L, L)]
                                 * mr[row, pl.ds(j * L, L)])
                p = jnp.where(lanes == r, jnp.sum(acc), p)
            outv[pl.ds(cbase + g * L, L)] = p
            return carry

        lax.fori_loop(0, RC // L, group, 0)

    for cp in copies:
        cp.wait()

    def bias_clip(g, carry):
        sl = pl.ds(g * L, L)
        p = outv[sl] + ubv[sl] + mbv[sl]
        outv[sl] = jnp.minimum(jnp.maximum(p, 0.0), 5.0)
        return carry

    lax.fori_loop(0, BPW // L, bias_clip, 0)
    pltpu.sync_copy(outv, out.at[pl.ds(base, BPW)])


def kernel(x1, x2, U, M, UB, MB):
    return _sc_predict(x1.astype(jnp.int32), x2.astype(jnp.int32),
                       U, M, UB.reshape(-1), MB.reshape(-1))
